# baseline (speedup 1.0000x reference)
import functools

import jax
import jax.numpy as jnp
from jax import lax
from jax.experimental import pallas as pl
from jax.experimental.pallas import tpu as pltpu
from jax.experimental.pallas import tpu_sc as plsc


def _take16(x, perm):
    dnums = lax.GatherDimensionNumbers(
        offset_dims=(), collapsed_slice_dims=(0,), start_index_map=(0,))
    return lax.gather(x, perm[:, None], dnums, (1,),
                      mode=lax.GatherScatterMode.PROMISE_IN_BOUNDS)


NUM_CORES = 2
NUM_SUBCORES = 16
LANES = 16
NUM_WORKERS = NUM_CORES * NUM_SUBCORES

BATCH = 16384
EMBED_DIM = 64
PAIR = 2 * EMBED_DIM
B_PER_W = BATCH // NUM_WORKERS
CH = 128
N_WAVES = B_PER_W // CH


def _body(h_hbm, r_hbm, t_hbm, ent2_hbm, rel2_hbm, out_hbm,
          h_idx, r_idx, t_idx, h_til, r_til, t_til,
          h_dst, r_dst, t_dst, out_v, sem):
    wid = lax.axis_index("s") * NUM_CORES + lax.axis_index("c")
    base = wid * B_PER_W

    pltpu.sync_copy(h_hbm.at[pl.ds(base, B_PER_W)], h_idx)
    pltpu.sync_copy(r_hbm.at[pl.ds(base, B_PER_W)], r_idx)
    pltpu.sync_copy(t_hbm.at[pl.ds(base, B_PER_W)], t_idx)

    def tid_body(g, c):
        sl = pl.ds(g * LANES, LANES)
        h_til[sl] = h_idx[sl] >> 1
        t_til[sl] = t_idx[sl] >> 1
        r_til[sl] = r_idx[sl] >> 1
        return c

    lax.fori_loop(0, B_PER_W // LANES, tid_body, 0)

    for w in range(N_WAVES):
        wb = w * CH
        sl = pl.ds(wb, CH)
        copies = [
            pltpu.async_copy(ent2_hbm.at[h_til.at[sl]], h_dst, sem),
            pltpu.async_copy(ent2_hbm.at[t_til.at[sl]], t_dst, sem),
            pltpu.async_copy(rel2_hbm.at[r_til.at[sl]], r_dst, sem),
        ]
        for cp in copies:
            cp.wait()

        def group_body(g, carry):
            lane = lax.iota(jnp.int32, LANES)
            gsl = pl.ds(wb + g * LANES, LANES)
            hsub = (h_idx[gsl] & 1) * EMBED_DIM
            tsub = (t_idx[gsl] & 1) * EMBED_DIM
            rsub = (r_idx[gsl] & 1) * EMBED_DIM
            vs = []
            for j in range(LANES):
                row = g * LANES + j
                hs = hsub[j]
                ts = tsub[j]
                rs = rsub[j]
                acc = None
                for k in range(EMBED_DIM // LANES):
                    hv = h_dst[row, pl.ds(hs + k * LANES, LANES)]
                    rv = r_dst[row, pl.ds(rs + k * LANES, LANES)]
                    tv = t_dst[row, pl.ds(ts + k * LANES, LANES)]
                    term = hv * rv - tv
                    acc = term if acc is None else acc + term
                vs.append(acc)
            for step in (1, 2, 4, 8):
                bit = (lane & step) != 0
                perm = lane ^ step
                nxt = []
                for a, b in zip(vs[0::2], vs[1::2]):
                    lo = jnp.where(bit, b, a)
                    hi = jnp.where(bit, a, b)
                    nxt.append(lo + _take16(hi, perm))
                vs = nxt
            out_v[gsl] = vs[0]
            return carry

        lax.fori_loop(0, CH // LANES, group_body, 0)

    pltpu.sync_copy(out_v, out_hbm.at[pl.ds(base, B_PER_W)])


def kernel(h, r, t, entity_emb, relation_emb):
    ent2 = entity_emb.reshape(-1, PAIR)
    rel2 = relation_emb.reshape(-1, PAIR)
    mesh = plsc.VectorSubcoreMesh(core_axis_name="c", subcore_axis_name="s")
    run = functools.partial(
        pl.kernel,
        mesh=mesh,
        compiler_params=pltpu.CompilerParams(use_tc_tiling_on_sc=True),
        out_type=jax.ShapeDtypeStruct((BATCH,), jnp.float32),
        scratch_types=[
            pltpu.VMEM((B_PER_W,), jnp.int32),
            pltpu.VMEM((B_PER_W,), jnp.int32),
            pltpu.VMEM((B_PER_W,), jnp.int32),
            pltpu.VMEM((B_PER_W,), jnp.int32),
            pltpu.VMEM((B_PER_W,), jnp.int32),
            pltpu.VMEM((B_PER_W,), jnp.int32),
            pltpu.VMEM((CH, PAIR), jnp.float32),
            pltpu.VMEM((CH, PAIR), jnp.float32),
            pltpu.VMEM((CH, PAIR), jnp.float32),
            pltpu.VMEM((B_PER_W,), jnp.float32),
            pltpu.SemaphoreType.DMA,
        ],
    )(_body)
    return run(h, r, t, ent2, rel2)
